# baseline (speedup 1.0000x reference)
import functools

import jax
import jax.numpy as jnp
from jax import lax
from jax.experimental import pallas as pl
from jax.experimental.pallas import tpu as pltpu
from jax.experimental.pallas import tpu_sc as plsc

B, S, H, MAXN = 16, 4096, 1024, 64

_NC, _NS, _L = 2, 16, 16
_NW = _NC * _NS
PAIRS = B * MAXN
PPW = PAIRS // _NW


def _index_kernel(nm_ref, first_ref, last_ref, scale_ref):
    ids = lax.broadcasted_iota(jnp.int32, (MAXN, 1), 0) + 1
    pos = lax.broadcasted_iota(jnp.int32, (MAXN, S), 1)
    for b in range(B):
        nm = nm_ref[pl.ds(b, 1), :]
        match = nm == ids
        first = jnp.min(jnp.where(match, pos, S), axis=1)
        last = jnp.max(jnp.where(match, pos, -1), axis=1)
        present = last >= 0
        sl = pl.ds(b * MAXN, MAXN)
        first_ref[sl] = jnp.where(present, first, 0) + b * S
        last_ref[sl] = jnp.where(present, last, 0) + b * S
        scale_ref[sl] = present.astype(jnp.float32)


def _compute_indices(nm):
    return pl.pallas_call(
        _index_kernel,
        out_shape=[
            jax.ShapeDtypeStruct((PAIRS,), jnp.int32),
            jax.ShapeDtypeStruct((PAIRS,), jnp.int32),
            jax.ShapeDtypeStruct((PAIRS,), jnp.float32),
        ],
    )(nm)


_PC = 8
_NCH = PPW // _PC


def _gather_body(table_hbm, first_hbm, last_hbm, scale_hbm, out_hbm,
                 fidx_v, lidx_v, scale_v, comb_v, gsem, wsem):
    wid = lax.axis_index("s") * _NC + lax.axis_index("c")
    pbase = wid * PPW
    pltpu.sync_copy(first_hbm.at[pl.ds(pbase, PPW)], fidx_v)
    pltpu.sync_copy(last_hbm.at[pl.ds(pbase, PPW)], lidx_v)
    pltpu.sync_copy(scale_hbm.at[pl.ds(pbase, PPW)], scale_v)

    m = scale_v[pl.ds(0, _L)]
    for g in range(1, PPW // _L):
        m = jnp.minimum(m, scale_v[pl.ds(g * _L, _L)])
    all_present = jnp.min(m)

    gathers = []
    for c in range(_NCH):
        row = pl.ds(c * _PC, _PC)
        gathers.append((
            pltpu.async_copy(table_hbm.at[fidx_v.at[row]],
                             comb_v.at[row, pl.ds(0, H)], gsem),
            pltpu.async_copy(table_hbm.at[lidx_v.at[row]],
                             comb_v.at[row, pl.ds(H, H)], gsem),
        ))
    writes = []
    for c in range(_NCH):
        row = pl.ds(c * _PC, _PC)
        gathers[c][0].wait()
        gathers[c][1].wait()

        @pl.when(all_present < 0.5)
        def _mask_rows(c=c):
            def col_body(k, carry):
                off = k * _L
                for r in range(c * _PC, (c + 1) * _PC):
                    srow = plsc.load_gather(
                        scale_v, [jnp.full((_L,), r, jnp.int32)])
                    comb_v[r, pl.ds(off, _L)] = (
                        comb_v[r, pl.ds(off, _L)] * srow)
                return carry
            lax.fori_loop(0, 2 * H // _L, col_body, 0)

        writes.append(pltpu.async_copy(
            comb_v.at[row], out_hbm.at[pl.ds(pbase + c * _PC, _PC)], wsem))
    for w in writes:
        w.wait()


@functools.cache
def _gather_rows():
    return pl.kernel(
        _gather_body,
        out_type=jax.ShapeDtypeStruct((PAIRS, 2 * H), jnp.float32),
        mesh=plsc.VectorSubcoreMesh(core_axis_name="c", subcore_axis_name="s"),
        compiler_params=pltpu.CompilerParams(needs_layout_passes=False),
        scratch_types=[
            pltpu.VMEM((PPW,), jnp.int32),
            pltpu.VMEM((PPW,), jnp.int32),
            pltpu.VMEM((PPW,), jnp.float32),
            pltpu.VMEM((PPW, 2 * H), jnp.float32),
            pltpu.SemaphoreType.DMA,
            pltpu.SemaphoreType.DMA,
        ],
    )


def kernel(input, attention_mask, question_mask, number_mask):
    nm = number_mask.astype(jnp.int32)
    first, last, scale = _compute_indices(nm)
    table = input.reshape(B * S, H)
    gathered = _gather_rows()(table, first, last, scale)
    return gathered.reshape(B, MAXN, 2 * H)


# speedup vs baseline: 1.0764x; 1.0764x over previous
import functools

import jax
import jax.numpy as jnp
from jax import lax
from jax.experimental import pallas as pl
from jax.experimental.pallas import tpu as pltpu
from jax.experimental.pallas import tpu_sc as plsc

B, S, H, MAXN = 16, 4096, 1024, 64

_NC, _NS, _L = 2, 16, 16
_NW = _NC * _NS
PAIRS = B * MAXN
PPW = PAIRS // _NW


_LANES = 128
_KB = S // _LANES
_SF = float(S)


def _index_kernel(nm_ref, first_ref, last_ref, scale_ref):
    ids = lax.broadcasted_iota(jnp.int32, (MAXN, 1), 0) + 1
    lanef = lax.broadcasted_iota(
        jnp.int32, (MAXN, _LANES), 1).astype(jnp.float32)
    for b in range(B):
        minacc = jnp.full((MAXN, _LANES), _SF, jnp.float32)
        maxacc = jnp.full((MAXN, _LANES), -1.0, jnp.float32)
        for k in range(_KB):
            nmk = nm_ref[pl.ds(b, 1), pl.ds(k * _LANES, _LANES)]
            matchk = nmk == ids
            posk = lanef + float(k * _LANES)
            maxacc = jnp.where(matchk, posk, maxacc)
            minacc = jnp.minimum(minacc, jnp.where(matchk, posk, _SF))
        first = jnp.min(minacc, axis=1).astype(jnp.int32)
        last = jnp.max(maxacc, axis=1).astype(jnp.int32)
        present = last >= 0
        sl = pl.ds(b * MAXN, MAXN)
        first_ref[sl] = jnp.where(present, first, 0) + b * S
        last_ref[sl] = jnp.where(present, last, 0) + b * S
        scale_ref[sl] = present.astype(jnp.float32)


def _compute_indices(nm):
    return pl.pallas_call(
        _index_kernel,
        out_shape=[
            jax.ShapeDtypeStruct((PAIRS,), jnp.int32),
            jax.ShapeDtypeStruct((PAIRS,), jnp.int32),
            jax.ShapeDtypeStruct((PAIRS,), jnp.float32),
        ],
    )(nm)


_PC = 8
_NCH = PPW // _PC


def _gather_body(table_hbm, first_hbm, last_hbm, scale_hbm, out_hbm,
                 fidx_v, lidx_v, scale_v, comb_v, gsem, wsem):
    wid = lax.axis_index("s") * _NC + lax.axis_index("c")
    pbase = wid * PPW
    pltpu.sync_copy(first_hbm.at[pl.ds(pbase, PPW)], fidx_v)
    pltpu.sync_copy(last_hbm.at[pl.ds(pbase, PPW)], lidx_v)
    pltpu.sync_copy(scale_hbm.at[pl.ds(pbase, PPW)], scale_v)

    m = scale_v[pl.ds(0, _L)]
    for g in range(1, PPW // _L):
        m = jnp.minimum(m, scale_v[pl.ds(g * _L, _L)])
    all_present = jnp.min(m)

    gathers = []
    for c in range(_NCH):
        row = pl.ds(c * _PC, _PC)
        gathers.append((
            pltpu.async_copy(table_hbm.at[fidx_v.at[row]],
                             comb_v.at[row, pl.ds(0, H)], gsem),
            pltpu.async_copy(table_hbm.at[lidx_v.at[row]],
                             comb_v.at[row, pl.ds(H, H)], gsem),
        ))
    writes = []
    for c in range(_NCH):
        row = pl.ds(c * _PC, _PC)
        gathers[c][0].wait()
        gathers[c][1].wait()

        @pl.when(all_present < 0.5)
        def _mask_rows(c=c):
            def col_body(k, carry):
                off = k * _L
                for r in range(c * _PC, (c + 1) * _PC):
                    srow = plsc.load_gather(
                        scale_v, [jnp.full((_L,), r, jnp.int32)])
                    comb_v[r, pl.ds(off, _L)] = (
                        comb_v[r, pl.ds(off, _L)] * srow)
                return carry
            lax.fori_loop(0, 2 * H // _L, col_body, 0)

        writes.append(pltpu.async_copy(
            comb_v.at[row], out_hbm.at[pl.ds(pbase + c * _PC, _PC)], wsem))
    for w in writes:
        w.wait()


@functools.cache
def _gather_rows():
    return pl.kernel(
        _gather_body,
        out_type=jax.ShapeDtypeStruct((PAIRS, 2 * H), jnp.float32),
        mesh=plsc.VectorSubcoreMesh(core_axis_name="c", subcore_axis_name="s"),
        compiler_params=pltpu.CompilerParams(needs_layout_passes=False),
        scratch_types=[
            pltpu.VMEM((PPW,), jnp.int32),
            pltpu.VMEM((PPW,), jnp.int32),
            pltpu.VMEM((PPW,), jnp.float32),
            pltpu.VMEM((PPW, 2 * H), jnp.float32),
            pltpu.SemaphoreType.DMA,
            pltpu.SemaphoreType.DMA,
        ],
    )


def kernel(input, attention_mask, question_mask, number_mask):
    nm = number_mask.astype(jnp.int32)
    first, last, scale = _compute_indices(nm)
    table = input.reshape(B * S, H)
    gathered = _gather_rows()(table, first, last, scale)
    return gathered.reshape(B, MAXN, 2 * H)


# speedup vs baseline: 1.0788x; 1.0022x over previous
import functools

import jax
import jax.numpy as jnp
from jax import lax
from jax.experimental import pallas as pl
from jax.experimental.pallas import tpu as pltpu
from jax.experimental.pallas import tpu_sc as plsc

B, S, H, MAXN = 16, 4096, 1024, 64

_NC, _NS, _L = 2, 16, 16
_NW = _NC * _NS
PAIRS = B * MAXN
PPW = PAIRS // _NW


_LANES = 128
_KB = S // _LANES
_SF = float(S)


def _index_kernel(nm_ref, first_ref, last_ref, scale_ref):
    ids = lax.broadcasted_iota(jnp.int32, (MAXN, 1), 0) + 1
    lanef = lax.broadcasted_iota(
        jnp.int32, (MAXN, _LANES), 1).astype(jnp.float32)
    for b in range(B):
        minacc = jnp.full((MAXN, _LANES), _SF, jnp.float32)
        maxacc = jnp.full((MAXN, _LANES), -1.0, jnp.float32)
        for k in range(_KB):
            nmk = nm_ref[pl.ds(b, 1), pl.ds(k * _LANES, _LANES)]
            matchk = nmk == ids
            posk = lanef + float(k * _LANES)
            maxacc = jnp.where(matchk, posk, maxacc)
            minacc = jnp.minimum(minacc, jnp.where(matchk, posk, _SF))
        first = jnp.min(minacc, axis=1).astype(jnp.int32)
        last = jnp.max(maxacc, axis=1).astype(jnp.int32)
        present = last >= 0
        sl = pl.ds(b * MAXN, MAXN)
        first_ref[sl] = jnp.where(present, first, 0) + b * S
        last_ref[sl] = jnp.where(present, last, 0) + b * S
        scale_ref[sl] = present.astype(jnp.float32)


def _compute_indices(nm):
    return pl.pallas_call(
        _index_kernel,
        out_shape=[
            jax.ShapeDtypeStruct((PAIRS,), jnp.int32),
            jax.ShapeDtypeStruct((PAIRS,), jnp.int32),
            jax.ShapeDtypeStruct((PAIRS,), jnp.float32),
        ],
    )(nm)


_PC = 8
_NCH = PPW // _PC


def _gather_body(table_hbm, first_hbm, last_hbm, scale_hbm, out_hbm,
                 fidx_v, lidx_v, scale_v, comb_v, gsem, wsem):
    wid = lax.axis_index("s") * _NC + lax.axis_index("c")
    pbase = wid * PPW
    pltpu.sync_copy(first_hbm.at[pl.ds(pbase, PPW)], fidx_v)
    pltpu.sync_copy(last_hbm.at[pl.ds(pbase, PPW)], lidx_v)
    pltpu.sync_copy(scale_hbm.at[pl.ds(pbase, PPW)], scale_v)

    m = scale_v[pl.ds(0, _L)]
    for g in range(1, PPW // _L):
        m = jnp.minimum(m, scale_v[pl.ds(g * _L, _L)])
    all_present = jnp.min(m)

    gathers = []
    for c in range(_NCH):
        row = pl.ds(c * _PC, _PC)
        gathers.append((
            pltpu.async_copy(table_hbm.at[fidx_v.at[row]],
                             comb_v.at[row, pl.ds(0, H)], gsem),
            pltpu.async_copy(table_hbm.at[lidx_v.at[row]],
                             comb_v.at[row, pl.ds(H, H)], gsem),
        ))
    writes = []
    for c in range(_NCH):
        row = pl.ds(c * _PC, _PC)
        gathers[c][0].wait()
        gathers[c][1].wait()

        @pl.when(all_present < 0.5)
        def _mask_rows(c=c):
            def col_body(k, carry):
                off = k * _L
                for r in range(c * _PC, (c + 1) * _PC):
                    srow = plsc.load_gather(
                        scale_v, [jnp.full((_L,), r, jnp.int32)])
                    comb_v[r, pl.ds(off, _L)] = (
                        comb_v[r, pl.ds(off, _L)] * srow)
                return carry
            lax.fori_loop(0, 2 * H // _L, col_body, 0)

        writes.append(pltpu.async_copy(
            comb_v.at[row], out_hbm.at[pl.ds(pbase + c * _PC, _PC)], wsem))
    for w in writes:
        w.wait()


@functools.cache
def _gather_rows():
    return pl.kernel(
        _gather_body,
        out_type=jax.ShapeDtypeStruct((PAIRS, 2 * H), jnp.float32),
        mesh=plsc.VectorSubcoreMesh(core_axis_name="c", subcore_axis_name="s"),
        compiler_params=pltpu.CompilerParams(
            needs_layout_passes=False, skip_device_barrier=True),
        scratch_types=[
            pltpu.VMEM((PPW,), jnp.int32),
            pltpu.VMEM((PPW,), jnp.int32),
            pltpu.VMEM((PPW,), jnp.float32),
            pltpu.VMEM((PPW, 2 * H), jnp.float32),
            pltpu.SemaphoreType.DMA,
            pltpu.SemaphoreType.DMA,
        ],
    )


def kernel(input, attention_mask, question_mask, number_mask):
    nm = number_mask.astype(jnp.int32)
    first, last, scale = _compute_indices(nm)
    table = input.reshape(B * S, H)
    gathered = _gather_rows()(table, first, last, scale)
    return gathered.reshape(B, MAXN, 2 * H)


# speedup vs baseline: 1.1218x; 1.0399x over previous
import functools

import jax
import jax.numpy as jnp
from jax import lax
from jax.experimental import pallas as pl
from jax.experimental.pallas import tpu as pltpu
from jax.experimental.pallas import tpu_sc as plsc

B, S, H, MAXN = 16, 4096, 1024, 64

_NC, _NS, _L = 2, 16, 16
_NW = _NC * _NS
PAIRS = B * MAXN
PPW = PAIRS // _NW


_LANES = 128
_KB = S // _LANES
_SF = float(S)


def _index_kernel(nm_ref, first_ref, last_ref, scale_ref):
    ids = (lax.broadcasted_iota(jnp.int32, (MAXN, 1), 0) + 1
           ).astype(jnp.bfloat16)
    lanef = lax.broadcasted_iota(
        jnp.int32, (MAXN, _LANES), 1).astype(jnp.float32)
    sent = jnp.bfloat16(_KB * 2)
    for b in range(B):
        minacc = jnp.full((MAXN, _LANES), sent, jnp.bfloat16)
        maxacc = jnp.full((MAXN, _LANES), -1.0, jnp.bfloat16)
        for k in range(_KB):
            nmk = nm_ref[pl.ds(b, 1), pl.ds(k * _LANES, _LANES)
                         ].astype(jnp.bfloat16)
            matchk = nmk == ids
            kb = jnp.bfloat16(k)
            maxacc = jnp.where(matchk, kb, maxacc)
            minacc = jnp.minimum(minacc, jnp.where(matchk, kb, sent))
        minf = minacc.astype(jnp.float32) * float(_LANES) + lanef
        maxf = maxacc.astype(jnp.float32) * float(_LANES) + lanef
        first = jnp.min(minf, axis=1).astype(jnp.int32)
        last = jnp.max(maxf, axis=1).astype(jnp.int32)
        present = last >= 0
        sl = pl.ds(b * MAXN, MAXN)
        first_ref[sl] = jnp.where(present, first, 0) + b * S
        last_ref[sl] = jnp.where(present, last, 0) + b * S
        scale_ref[sl] = present.astype(jnp.float32)


def _compute_indices(nm):
    return pl.pallas_call(
        _index_kernel,
        out_shape=[
            jax.ShapeDtypeStruct((PAIRS,), jnp.int32),
            jax.ShapeDtypeStruct((PAIRS,), jnp.int32),
            jax.ShapeDtypeStruct((PAIRS,), jnp.float32),
        ],
    )(nm)


_PC = 8
_NCH = PPW // _PC


def _gather_body(table_hbm, first_hbm, last_hbm, scale_hbm, out_hbm,
                 fidx_v, lidx_v, scale_v, comb_v, gsem, wsem):
    wid = lax.axis_index("s") * _NC + lax.axis_index("c")
    pbase = wid * PPW
    pltpu.sync_copy(first_hbm.at[pl.ds(pbase, PPW)], fidx_v)
    pltpu.sync_copy(last_hbm.at[pl.ds(pbase, PPW)], lidx_v)
    pltpu.sync_copy(scale_hbm.at[pl.ds(pbase, PPW)], scale_v)

    m = scale_v[pl.ds(0, _L)]
    for g in range(1, PPW // _L):
        m = jnp.minimum(m, scale_v[pl.ds(g * _L, _L)])
    all_present = jnp.min(m)

    gathers = []
    for c in range(_NCH):
        row = pl.ds(c * _PC, _PC)
        gathers.append((
            pltpu.async_copy(table_hbm.at[fidx_v.at[row]],
                             comb_v.at[row, pl.ds(0, H)], gsem),
            pltpu.async_copy(table_hbm.at[lidx_v.at[row]],
                             comb_v.at[row, pl.ds(H, H)], gsem),
        ))
    writes = []
    for c in range(_NCH):
        row = pl.ds(c * _PC, _PC)
        gathers[c][0].wait()
        gathers[c][1].wait()

        @pl.when(all_present < 0.5)
        def _mask_rows(c=c):
            def col_body(k, carry):
                off = k * _L
                for r in range(c * _PC, (c + 1) * _PC):
                    srow = plsc.load_gather(
                        scale_v, [jnp.full((_L,), r, jnp.int32)])
                    comb_v[r, pl.ds(off, _L)] = (
                        comb_v[r, pl.ds(off, _L)] * srow)
                return carry
            lax.fori_loop(0, 2 * H // _L, col_body, 0)

        writes.append(pltpu.async_copy(
            comb_v.at[row], out_hbm.at[pl.ds(pbase + c * _PC, _PC)], wsem))
    for w in writes:
        w.wait()


@functools.cache
def _gather_rows():
    return pl.kernel(
        _gather_body,
        out_type=jax.ShapeDtypeStruct((PAIRS, 2 * H), jnp.float32),
        mesh=plsc.VectorSubcoreMesh(core_axis_name="c", subcore_axis_name="s"),
        compiler_params=pltpu.CompilerParams(
            needs_layout_passes=False, skip_device_barrier=True),
        scratch_types=[
            pltpu.VMEM((PPW,), jnp.int32),
            pltpu.VMEM((PPW,), jnp.int32),
            pltpu.VMEM((PPW,), jnp.float32),
            pltpu.VMEM((PPW, 2 * H), jnp.float32),
            pltpu.SemaphoreType.DMA,
            pltpu.SemaphoreType.DMA,
        ],
    )


def kernel(input, attention_mask, question_mask, number_mask):
    nm = number_mask.astype(jnp.int32)
    first, last, scale = _compute_indices(nm)
    table = input.reshape(B * S, H)
    gathered = _gather_rows()(table, first, last, scale)
    return gathered.reshape(B, MAXN, 2 * H)


# speedup vs baseline: 1.1477x; 1.0231x over previous
import functools

import jax
import jax.numpy as jnp
from jax import lax
from jax.experimental import pallas as pl
from jax.experimental.pallas import tpu as pltpu
from jax.experimental.pallas import tpu_sc as plsc

B, S, H, MAXN = 16, 4096, 1024, 64

_NC, _NS, _L = 2, 16, 16
_NW = _NC * _NS
PAIRS = B * MAXN
PPW = PAIRS // _NW


_LANES = 128
_KB = S // _LANES
_SF = float(S)


def _index_kernel(nm_ref, first_ref, last_ref, scale_ref):
    ids = (lax.broadcasted_iota(jnp.int32, (MAXN, 1), 0) + 1
           ).astype(jnp.bfloat16)
    lanef = lax.broadcasted_iota(
        jnp.int32, (MAXN, _LANES), 1).astype(jnp.float32)
    sent = jnp.bfloat16(_KB * 2)
    for b in range(B):
        minacc = jnp.full((MAXN, _LANES), sent, jnp.bfloat16)
        maxacc = jnp.full((MAXN, _LANES), -1.0, jnp.bfloat16)
        for k in range(_KB):
            nmk = nm_ref[pl.ds(b, 1), pl.ds(k * _LANES, _LANES)
                         ].astype(jnp.bfloat16)
            matchk = nmk == ids
            kb = jnp.bfloat16(k)
            maxacc = jnp.where(matchk, kb, maxacc)
            minacc = jnp.minimum(minacc, jnp.where(matchk, kb, sent))
        minf = minacc.astype(jnp.float32) * float(_LANES) + lanef
        maxf = maxacc.astype(jnp.float32) * float(_LANES) + lanef
        first = jnp.min(minf, axis=1).astype(jnp.int32)
        last = jnp.max(maxf, axis=1).astype(jnp.int32)
        present = last >= 0
        sl = pl.ds(b * MAXN, MAXN)
        first_ref[sl] = jnp.where(present, first, 0) + b * S
        last_ref[sl] = jnp.where(present, last, 0) + b * S
        scale_ref[sl] = present.astype(jnp.float32)


def _compute_indices(nm):
    return pl.pallas_call(
        _index_kernel,
        out_shape=[
            jax.ShapeDtypeStruct((PAIRS,), jnp.int32),
            jax.ShapeDtypeStruct((PAIRS,), jnp.int32),
            jax.ShapeDtypeStruct((PAIRS,), jnp.float32),
        ],
    )(nm)


_PC = 8
_NCH = PPW // _PC


def _gather_body(table_hbm, first_hbm, last_hbm, scale_hbm, out_hbm,
                 fidx_v, lidx_v, scale_v, comb_v, gsem, wsem):
    wid = lax.axis_index("s") * _NC + lax.axis_index("c")
    pbase = wid * PPW
    i1 = pltpu.async_copy(first_hbm.at[pl.ds(pbase, PPW)], fidx_v, wsem)
    i2 = pltpu.async_copy(last_hbm.at[pl.ds(pbase, PPW)], lidx_v, wsem)
    i3 = pltpu.async_copy(scale_hbm.at[pl.ds(pbase, PPW)], scale_v, wsem)
    i1.wait()
    i2.wait()
    i3.wait()

    gathers = []
    for c in range(_NCH):
        row = pl.ds(c * _PC, _PC)
        gathers.append((
            pltpu.async_copy(table_hbm.at[fidx_v.at[row]],
                             comb_v.at[row, pl.ds(0, H)], gsem),
            pltpu.async_copy(table_hbm.at[lidx_v.at[row]],
                             comb_v.at[row, pl.ds(H, H)], gsem),
        ))

    m = scale_v[pl.ds(0, _L)]
    for g in range(1, PPW // _L):
        m = jnp.minimum(m, scale_v[pl.ds(g * _L, _L)])
    all_present = jnp.min(m)
    writes = []
    for c in range(_NCH):
        row = pl.ds(c * _PC, _PC)
        gathers[c][0].wait()
        gathers[c][1].wait()

        @pl.when(all_present < 0.5)
        def _mask_rows(c=c):
            def col_body(k, carry):
                off = k * _L
                for r in range(c * _PC, (c + 1) * _PC):
                    srow = plsc.load_gather(
                        scale_v, [jnp.full((_L,), r, jnp.int32)])
                    comb_v[r, pl.ds(off, _L)] = (
                        comb_v[r, pl.ds(off, _L)] * srow)
                return carry
            lax.fori_loop(0, 2 * H // _L, col_body, 0)

        writes.append(pltpu.async_copy(
            comb_v.at[row], out_hbm.at[pl.ds(pbase + c * _PC, _PC)], wsem))
    for w in writes:
        w.wait()


@functools.cache
def _gather_rows():
    return pl.kernel(
        _gather_body,
        out_type=jax.ShapeDtypeStruct((PAIRS, 2 * H), jnp.float32),
        mesh=plsc.VectorSubcoreMesh(core_axis_name="c", subcore_axis_name="s"),
        compiler_params=pltpu.CompilerParams(
            needs_layout_passes=False, skip_device_barrier=True),
        scratch_types=[
            pltpu.VMEM((PPW,), jnp.int32),
            pltpu.VMEM((PPW,), jnp.int32),
            pltpu.VMEM((PPW,), jnp.float32),
            pltpu.VMEM((PPW, 2 * H), jnp.float32),
            pltpu.SemaphoreType.DMA,
            pltpu.SemaphoreType.DMA,
        ],
    )


def kernel(input, attention_mask, question_mask, number_mask):
    nm = number_mask.astype(jnp.int32)
    first, last, scale = _compute_indices(nm)
    table = input.reshape(B * S, H)
    gathered = _gather_rows()(table, first, last, scale)
    return gathered.reshape(B, MAXN, 2 * H)
